# baseline (speedup 1.0000x reference)
import functools

import jax
import jax.numpy as jnp
from jax import lax
from jax.experimental import pallas as pl
from jax.experimental.pallas import tpu as pltpu
from jax.experimental.pallas import tpu_sc as plsc

N_EDGES_K = 320000
D_K = 128
N_NODES_K = 10000
N_PAD_K = 10240

NC = 2
NS = 16
NW = NC * NS

EDGES_PER_TILE = N_EDGES_K // NW
BLK = 128
NF = EDGES_PER_TILE // BLK
TAIL = EDGES_PER_TILE - NF * BLK
ROWS_PER_TILE = N_PAD_K // NS
ZROWS = 128


def _sc_partial_sums(x, index):
    mesh = plsc.VectorSubcoreMesh(
        core_axis_name="c", subcore_axis_name="s", num_cores=NC,
        num_subcores=NS)

    @functools.partial(
        pl.kernel,
        out_type=jax.ShapeDtypeStruct((NC * N_PAD_K, D_K), jnp.float32),
        mesh=mesh,
        scratch_types=[
            pltpu.VMEM((2, BLK, D_K), jnp.float32),
            pltpu.VMEM((2, BLK), jnp.int32),
            pltpu.VMEM((TAIL, D_K), jnp.float32),
            pltpu.VMEM((TAIL,), jnp.int32),
            pltpu.SemaphoreType.DMA,
            pltpu.SemaphoreType.DMA,
            pltpu.VMEM_SHARED((N_PAD_K, D_K), jnp.float32),
        ],
    )
    def sc_kernel(x_hbm, idx_hbm, part_hbm, rows_v, idx_v, rows_t, idx_t,
                  sem0, sem1, acc_sh):
        c = lax.axis_index("c")
        s = lax.axis_index("s")
        wid = c * NS + s
        base = wid * EDGES_PER_TILE

        zvec = jnp.zeros((16,), jnp.float32)

        def zero_row(i):
            for k in range(D_K // 16):
                rows_v[0, i, pl.ds(k * 16, 16)] = zvec

        pl.loop(0, ZROWS)(zero_row)

        def zero_acc(j):
            pltpu.sync_copy(
                rows_v.at[0],
                acc_sh.at[pl.ds(s * ROWS_PER_TILE + j * ZROWS, ZROWS)])

        pl.loop(0, ROWS_PER_TILE // ZROWS)(zero_acc)
        plsc.subcore_barrier()

        sems = (sem0, sem1)

        def start_copy(g, b):
            e0 = base + g * BLK
            pltpu.async_copy(idx_hbm.at[pl.ds(e0, BLK)], idx_v.at[b],
                             sems[b])
            pltpu.async_copy(x_hbm.at[pl.ds(e0, BLK)], rows_v.at[b],
                             sems[b])

        def wait_copy(g, b):
            e0 = base + g * BLK
            pltpu.make_async_copy(idx_hbm.at[pl.ds(e0, BLK)], idx_v.at[b],
                                  sems[b]).wait()
            pltpu.make_async_copy(x_hbm.at[pl.ds(e0, BLK)], rows_v.at[b],
                                  sems[b]).wait()

        def scatter_block(b):
            pltpu.sync_copy(rows_v.at[b], acc_sh.at[idx_v.at[b]], add=True)

        start_copy(0, 0)

        def body(h):
            g0 = 2 * h
            start_copy(g0 + 1, 1)
            wait_copy(g0, 0)
            scatter_block(0)
            start_copy(g0 + 2, 0)
            wait_copy(g0 + 1, 1)
            scatter_block(1)

        pl.loop(0, (NF - 2) // 2)(body)
        e_t = base + NF * BLK
        start_copy(NF - 1, 1)
        wait_copy(NF - 2, 0)
        scatter_block(0)
        pltpu.async_copy(idx_hbm.at[pl.ds(e_t, TAIL)], idx_t, sem0)
        pltpu.async_copy(x_hbm.at[pl.ds(e_t, TAIL)], rows_t, sem0)
        wait_copy(NF - 1, 1)
        scatter_block(1)
        pltpu.make_async_copy(idx_hbm.at[pl.ds(e_t, TAIL)], idx_t,
                              sem0).wait()
        pltpu.make_async_copy(x_hbm.at[pl.ds(e_t, TAIL)], rows_t,
                              sem0).wait()
        pltpu.sync_copy(rows_t, acc_sh.at[idx_t], add=True)
        plsc.subcore_barrier()

        out_row = c * N_PAD_K + s * ROWS_PER_TILE
        pltpu.sync_copy(acc_sh.at[pl.ds(s * ROWS_PER_TILE, ROWS_PER_TILE)],
                        part_hbm.at[pl.ds(out_row, ROWS_PER_TILE)])

    return sc_kernel(x, index)


def _merge_body(a_ref, b_ref, o_ref):
    o_ref[...] = a_ref[...] + b_ref[...]


def _merge_partials(part):
    blk = 1024
    grid = (N_NODES_K + blk - 1) // blk
    off = N_PAD_K // blk
    return pl.pallas_call(
        _merge_body,
        out_shape=jax.ShapeDtypeStruct((N_NODES_K, D_K), jnp.float32),
        grid=(grid,),
        in_specs=[
            pl.BlockSpec((blk, D_K), lambda i: (i, 0)),
            pl.BlockSpec((blk, D_K), lambda i: (i + off, 0)),
        ],
        out_specs=pl.BlockSpec((blk, D_K), lambda i: (i, 0)),
    )(part, part)


def kernel(x, index):
    part = _sc_partial_sums(x, index)
    return _merge_partials(part)


# speedup vs baseline: 1.0138x; 1.0138x over previous
import functools

import jax
import jax.numpy as jnp
from jax import lax
from jax.experimental import pallas as pl
from jax.experimental.pallas import tpu as pltpu
from jax.experimental.pallas import tpu_sc as plsc

N_EDGES_K = 320000
D_K = 128
N_NODES_K = 10000
N_PAD_K = 10240

NC = 2
NS = 16
NW = NC * NS

EDGES_PER_TILE = N_EDGES_K // NW
BLK = 128
NF = EDGES_PER_TILE // BLK
TAIL = EDGES_PER_TILE - NF * BLK
ROWS_PER_TILE = N_PAD_K // NS
ZROWS = 128


def _sc_partial_sums(x, index):
    mesh = plsc.VectorSubcoreMesh(
        core_axis_name="c", subcore_axis_name="s", num_cores=NC,
        num_subcores=NS)

    @functools.partial(
        pl.kernel,
        out_type=jax.ShapeDtypeStruct((NC * N_PAD_K, D_K), jnp.float32),
        mesh=mesh,
        scratch_types=[
            pltpu.VMEM((2, BLK, D_K), jnp.float32),
            pltpu.VMEM((2, BLK), jnp.int32),
            pltpu.VMEM((TAIL, D_K), jnp.float32),
            pltpu.VMEM((TAIL,), jnp.int32),
            pltpu.SemaphoreType.DMA,
            pltpu.SemaphoreType.DMA,
            pltpu.VMEM_SHARED((N_PAD_K, D_K), jnp.float32),
        ],
    )
    def sc_kernel(x_hbm, idx_hbm, part_hbm, rows_v, idx_v, rows_t, idx_t,
                  sem0, sem1, acc_sh):
        c = lax.axis_index("c")
        s = lax.axis_index("s")
        wid = c * NS + s
        base = wid * EDGES_PER_TILE

        zvec = jnp.zeros((16,), jnp.float32)

        def zero_row(i):
            for k in range(D_K // 16):
                rows_v[0, i, pl.ds(k * 16, 16)] = zvec

        pl.loop(0, ZROWS)(zero_row)

        def zero_acc(j):
            pltpu.sync_copy(
                rows_v.at[0],
                acc_sh.at[pl.ds(s * ROWS_PER_TILE + j * ZROWS, ZROWS)])

        pl.loop(0, ROWS_PER_TILE // ZROWS)(zero_acc)
        plsc.subcore_barrier()

        sems = (sem0, sem1)

        def start_copy(g, b):
            e0 = base + g * BLK
            pltpu.async_copy(idx_hbm.at[pl.ds(e0, BLK)], idx_v.at[b],
                             sems[b])
            pltpu.async_copy(x_hbm.at[pl.ds(e0, BLK)], rows_v.at[b],
                             sems[b])

        def wait_copy(g, b):
            e0 = base + g * BLK
            pltpu.make_async_copy(idx_hbm.at[pl.ds(e0, BLK)], idx_v.at[b],
                                  sems[b]).wait()
            pltpu.make_async_copy(x_hbm.at[pl.ds(e0, BLK)], rows_v.at[b],
                                  sems[b]).wait()

        def scatter_block(b):
            pltpu.sync_copy(rows_v.at[b], acc_sh.at[idx_v.at[b]], add=True)

        start_copy(0, 0)

        def body(h):
            g0 = 2 * h
            start_copy(g0 + 1, 1)
            wait_copy(g0, 0)
            scatter_block(0)
            start_copy(g0 + 2, 0)
            wait_copy(g0 + 1, 1)
            scatter_block(1)

        pl.loop(0, (NF - 2) // 2)(body)
        e_t = base + NF * BLK
        start_copy(NF - 1, 1)
        wait_copy(NF - 2, 0)
        scatter_block(0)
        pltpu.async_copy(idx_hbm.at[pl.ds(e_t, TAIL)], idx_t, sem0)
        pltpu.async_copy(x_hbm.at[pl.ds(e_t, TAIL)], rows_t, sem0)
        wait_copy(NF - 1, 1)
        scatter_block(1)
        pltpu.make_async_copy(idx_hbm.at[pl.ds(e_t, TAIL)], idx_t,
                              sem0).wait()
        pltpu.make_async_copy(x_hbm.at[pl.ds(e_t, TAIL)], rows_t,
                              sem0).wait()
        pltpu.sync_copy(rows_t, acc_sh.at[idx_t], add=True)
        plsc.subcore_barrier()

        out_row = c * N_PAD_K + s * ROWS_PER_TILE
        pltpu.sync_copy(acc_sh.at[pl.ds(s * ROWS_PER_TILE, ROWS_PER_TILE)],
                        part_hbm.at[pl.ds(out_row, ROWS_PER_TILE)])

    return sc_kernel(x, index)


def _merge_body(a_ref, b_ref, o_ref):
    o_ref[...] = a_ref[...] + b_ref[...]


def _merge_partials(part):
    blk = 2048
    grid = (N_NODES_K + blk - 1) // blk
    off = N_PAD_K // blk
    return pl.pallas_call(
        _merge_body,
        out_shape=jax.ShapeDtypeStruct((N_NODES_K, D_K), jnp.float32),
        grid=(grid,),
        in_specs=[
            pl.BlockSpec((blk, D_K), lambda i: (i, 0)),
            pl.BlockSpec((blk, D_K), lambda i: (i + off, 0)),
        ],
        out_specs=pl.BlockSpec((blk, D_K), lambda i: (i, 0)),
    )(part, part)


def kernel(x, index):
    part = _sc_partial_sums(x, index)
    return _merge_partials(part)


# speedup vs baseline: 1.0204x; 1.0065x over previous
import functools

import jax
import jax.numpy as jnp
from jax import lax
from jax.experimental import pallas as pl
from jax.experimental.pallas import tpu as pltpu
from jax.experimental.pallas import tpu_sc as plsc

N_EDGES_K = 320000
D_K = 128
N_NODES_K = 10000
N_PAD_K = 10240

NC = 2
NS = 16
NW = NC * NS

EDGES_PER_TILE = N_EDGES_K // NW
BLK = 128
NF = EDGES_PER_TILE // BLK
TAIL = EDGES_PER_TILE - NF * BLK
ROWS_PER_TILE = N_PAD_K // NS
ZROWS = 128


def _sc_partial_sums(x, index):
    mesh = plsc.VectorSubcoreMesh(
        core_axis_name="c", subcore_axis_name="s", num_cores=NC,
        num_subcores=NS)

    @functools.partial(
        pl.kernel,
        out_type=jax.ShapeDtypeStruct((NC * N_PAD_K, D_K), jnp.float32),
        mesh=mesh,
        scratch_types=[
            pltpu.VMEM((2, BLK, D_K), jnp.float32),
            pltpu.VMEM((2, BLK), jnp.int32),
            pltpu.VMEM((TAIL, D_K), jnp.float32),
            pltpu.VMEM((TAIL,), jnp.int32),
            pltpu.SemaphoreType.DMA,
            pltpu.SemaphoreType.DMA,
            pltpu.VMEM_SHARED((N_PAD_K, D_K), jnp.float32),
        ],
    )
    def sc_kernel(x_hbm, idx_hbm, part_hbm, rows_v, idx_v, rows_t, idx_t,
                  sem0, sem1, acc_sh):
        c = lax.axis_index("c")
        s = lax.axis_index("s")
        wid = c * NS + s
        base = wid * EDGES_PER_TILE

        zvec = jnp.zeros((16,), jnp.float32)

        def zero_row(i):
            for k in range(D_K // 16):
                rows_v[0, i, pl.ds(k * 16, 16)] = zvec

        pl.loop(0, ZROWS)(zero_row)

        def zero_acc(j):
            pltpu.sync_copy(
                rows_v.at[0],
                acc_sh.at[pl.ds(s * ROWS_PER_TILE + j * ZROWS, ZROWS)])

        pl.loop(0, ROWS_PER_TILE // ZROWS)(zero_acc)
        plsc.subcore_barrier()

        sems = (sem0, sem1)

        def start_copy(g, b):
            e0 = base + g * BLK
            pltpu.async_copy(idx_hbm.at[pl.ds(e0, BLK)], idx_v.at[b],
                             sems[b])
            pltpu.async_copy(x_hbm.at[pl.ds(e0, BLK)], rows_v.at[b],
                             sems[b])

        def wait_copy(g, b):
            e0 = base + g * BLK
            pltpu.make_async_copy(idx_hbm.at[pl.ds(e0, BLK)], idx_v.at[b],
                                  sems[b]).wait()
            pltpu.make_async_copy(x_hbm.at[pl.ds(e0, BLK)], rows_v.at[b],
                                  sems[b]).wait()

        def scatter_block(b):
            pltpu.sync_copy(rows_v.at[b], acc_sh.at[idx_v.at[b]], add=True)

        start_copy(0, 0)

        def body(h):
            g0 = 2 * h
            start_copy(g0 + 1, 1)
            wait_copy(g0, 0)
            scatter_block(0)
            start_copy(g0 + 2, 0)
            wait_copy(g0 + 1, 1)
            scatter_block(1)

        pl.loop(0, (NF - 2) // 2)(body)
        e_t = base + NF * BLK
        start_copy(NF - 1, 1)
        wait_copy(NF - 2, 0)
        scatter_block(0)
        pltpu.async_copy(idx_hbm.at[pl.ds(e_t, TAIL)], idx_t, sem0)
        pltpu.async_copy(x_hbm.at[pl.ds(e_t, TAIL)], rows_t, sem0)
        wait_copy(NF - 1, 1)
        scatter_block(1)
        pltpu.make_async_copy(idx_hbm.at[pl.ds(e_t, TAIL)], idx_t,
                              sem0).wait()
        pltpu.make_async_copy(x_hbm.at[pl.ds(e_t, TAIL)], rows_t,
                              sem0).wait()
        pltpu.sync_copy(rows_t, acc_sh.at[idx_t], add=True)
        plsc.subcore_barrier()

        out_row = c * N_PAD_K + s * ROWS_PER_TILE
        pltpu.sync_copy(acc_sh.at[pl.ds(s * ROWS_PER_TILE, ROWS_PER_TILE)],
                        part_hbm.at[pl.ds(out_row, ROWS_PER_TILE)])

    return sc_kernel(x, index)


def _merge_body(a_ref, b_ref, o_ref):
    o_ref[...] = a_ref[...] + b_ref[...]


def _merge_partials(part):
    blk = 2560
    grid = (N_NODES_K + blk - 1) // blk
    off = N_PAD_K // blk
    return pl.pallas_call(
        _merge_body,
        out_shape=jax.ShapeDtypeStruct((N_NODES_K, D_K), jnp.float32),
        grid=(grid,),
        in_specs=[
            pl.BlockSpec((blk, D_K), lambda i: (i, 0)),
            pl.BlockSpec((blk, D_K), lambda i: (i + off, 0)),
        ],
        out_specs=pl.BlockSpec((blk, D_K), lambda i: (i, 0)),
    )(part, part)


def kernel(x, index):
    part = _sc_partial_sums(x, index)
    return _merge_partials(part)
